# baseline (speedup 1.0000x reference)
import functools

import jax
import jax.numpy as jnp
from jax import lax
from jax.experimental import pallas as pl
from jax.experimental.pallas import tpu as pltpu
from jax.experimental.pallas import tpu_sc as plsc

MAXLEN = 200
EMBED = 64
BATCH = 4096
NC, NS = 2, 16
NW = NC * NS
ROWS_W = BATCH // NW
CHUNK = 40
NH = MAXLEN // CHUNK
NCHUNKS = ROWS_W * NH
NBUF = 8
LANES = 16


def _body(x_hbm, tok_hbm, pos_hbm, out_hbm, idx_v, pos_v, buf, gsem, ssem):
  wid = lax.axis_index("s") * NC + lax.axis_index("c")
  cbase = wid * NCHUNKS

  pltpu.sync_copy(x_hbm.at[pl.ds(cbase, NCHUNKS)], idx_v)
  pltpu.sync_copy(pos_hbm, pos_v)

  def g_start(c, b):
    pltpu.async_copy(tok_hbm.at[idx_v.at[c]], buf.at[b], gsem.at[b])

  def g_wait(c, b):
    pltpu.make_async_copy(tok_hbm.at[idx_v.at[c]], buf.at[b], gsem.at[b]).wait()

  def s_start(c, b):
    dst = out_hbm.at[pl.ds((cbase + c) * CHUNK, CHUNK)]
    pltpu.async_copy(buf.at[b], dst, ssem.at[b])

  def s_wait(c, b):
    dst = out_hbm.at[pl.ds((cbase + c) * CHUNK, CHUNK)]
    pltpu.make_async_copy(buf.at[b], dst, ssem.at[b]).wait()

  for b in range(NBUF - 1):
    g_start(b, b)

  @pl.loop(0, NCHUNKS // NBUF)
  def _(t):
    for b in range(NBUF):
      c = t * NBUF + b
      bn = (b + NBUF - 1) % NBUF
      @pl.when(c + NBUF - 1 < NCHUNKS)
      def _():
        @pl.when(c >= 1)
        def _():
          s_wait(c - 1, bn)
        g_start(c + NBUF - 1, bn)

      g_wait(c, b)
      poff = (c % NH) * CHUNK
      bref = buf.at[b]

      @pl.loop(0, CHUNK)
      def _(i):
        for j in range(EMBED // LANES):
          sl = pl.ds(j * LANES, LANES)
          bref[i, sl] = bref[i, sl] + pos_v[poff + i, sl]

      s_start(c, b)

  for b in range(NBUF):
    s_wait(NCHUNKS - NBUF + b, b)


@functools.partial(
    pl.kernel,
    out_type=jax.ShapeDtypeStruct((BATCH * MAXLEN, EMBED), jnp.float32),
    mesh=plsc.VectorSubcoreMesh(
        core_axis_name="c", subcore_axis_name="s",
        num_cores=NC, num_subcores=NS),
    scratch_types=[
        pltpu.VMEM((NCHUNKS, CHUNK), jnp.int32),
        pltpu.VMEM((MAXLEN, EMBED), jnp.float32),
        pltpu.VMEM((NBUF, CHUNK, EMBED), jnp.float32),
        pltpu.SemaphoreType.DMA((NBUF,)),
        pltpu.SemaphoreType.DMA((NBUF,)),
    ],
    compiler_params=pltpu.CompilerParams(use_tc_tiling_on_sc=False),
)
def _emb(x_hbm, tok_hbm, pos_hbm, out_hbm, idx_v, pos_v, buf, gsem, ssem):
  _body(x_hbm, tok_hbm, pos_hbm, out_hbm, idx_v, pos_v, buf, gsem, ssem)


@jax.jit
def kernel(x, token_table, pos_table):
  x2 = x.astype(jnp.int32).reshape(BATCH * NH, CHUNK)
  out = _emb(x2, token_table, pos_table)
  return out.reshape(BATCH, MAXLEN, EMBED)


# speedup vs baseline: 1.1100x; 1.1100x over previous
import functools

import jax
import jax.numpy as jnp
from jax import lax
from jax.experimental import pallas as pl
from jax.experimental.pallas import tpu as pltpu
from jax.experimental.pallas import tpu_sc as plsc

MAXLEN = 200
EMBED = 64
BATCH = 4096
NC, NS = 2, 16
NW = NC * NS
ROWS_W = BATCH // NW
CHUNK = 128
NCHUNKS = ROWS_W * MAXLEN // CHUNK
NBUF = 4
LANES = 16


def _body(x_hbm, tok_hbm, pos_hbm, out_hbm, idx_v, pos_v, buf, gsem, ssem):
  wid = lax.axis_index("s") * NC + lax.axis_index("c")
  cbase = wid * NCHUNKS

  pltpu.sync_copy(x_hbm.at[pl.ds(cbase, NCHUNKS)], idx_v)
  pltpu.sync_copy(pos_hbm, pos_v)

  def g_start(c, b):
    pltpu.async_copy(tok_hbm.at[idx_v.at[c]], buf.at[b], gsem.at[b])

  def g_wait(c, b):
    pltpu.make_async_copy(tok_hbm.at[idx_v.at[c]], buf.at[b], gsem.at[b]).wait()

  def s_start(c, b):
    dst = out_hbm.at[pl.ds((cbase + c) * CHUNK, CHUNK)]
    pltpu.async_copy(buf.at[b], dst, ssem.at[b])

  def s_wait(c, b):
    dst = out_hbm.at[pl.ds((cbase + c) * CHUNK, CHUNK)]
    pltpu.make_async_copy(buf.at[b], dst, ssem.at[b]).wait()

  for b in range(NBUF - 1):
    g_start(b, b)

  @pl.loop(0, NCHUNKS // NBUF)
  def _(t):
    for b in range(NBUF):
      c = t * NBUF + b
      bn = (b + NBUF - 1) % NBUF
      @pl.when(c + NBUF - 1 < NCHUNKS)
      def _():
        @pl.when(c >= 1)
        def _():
          s_wait(c - 1, bn)
        g_start(c + NBUF - 1, bn)

      g_wait(c, b)
      poff = (c * CHUNK) % MAXLEN
      n1 = jnp.minimum(MAXLEN - poff, CHUNK)
      bref = buf.at[b]

      @pl.loop(0, n1)
      def _(i):
        for j in range(EMBED // LANES):
          sl = pl.ds(j * LANES, LANES)
          plsc.addupdate(bref.at[i, sl], pos_v[poff + i, sl])

      @pl.loop(n1, CHUNK)
      def _(i):
        for j in range(EMBED // LANES):
          sl = pl.ds(j * LANES, LANES)
          plsc.addupdate(bref.at[i, sl], pos_v[i - n1, sl])

      s_start(c, b)

  for b in range(NBUF):
    s_wait(NCHUNKS - NBUF + b, b)


@functools.partial(
    pl.kernel,
    out_type=jax.ShapeDtypeStruct((BATCH * MAXLEN, EMBED), jnp.float32),
    mesh=plsc.VectorSubcoreMesh(
        core_axis_name="c", subcore_axis_name="s",
        num_cores=NC, num_subcores=NS),
    scratch_types=[
        pltpu.VMEM((NCHUNKS, CHUNK), jnp.int32),
        pltpu.VMEM((MAXLEN, EMBED), jnp.float32),
        pltpu.VMEM((NBUF, CHUNK, EMBED), jnp.float32),
        pltpu.SemaphoreType.DMA((NBUF,)),
        pltpu.SemaphoreType.DMA((NBUF,)),
    ],
    compiler_params=pltpu.CompilerParams(use_tc_tiling_on_sc=False),
)
def _emb(x_hbm, tok_hbm, pos_hbm, out_hbm, idx_v, pos_v, buf, gsem, ssem):
  _body(x_hbm, tok_hbm, pos_hbm, out_hbm, idx_v, pos_v, buf, gsem, ssem)


@jax.jit
def kernel(x, token_table, pos_table):
  x2 = x.astype(jnp.int32).reshape(-1, CHUNK)
  out = _emb(x2, token_table, pos_table)
  return out.reshape(BATCH, MAXLEN, EMBED)


# speedup vs baseline: 1.3511x; 1.2172x over previous
import functools

import jax
import jax.numpy as jnp
from jax import lax
from jax.experimental import pallas as pl
from jax.experimental.pallas import tpu as pltpu
from jax.experimental.pallas import tpu_sc as plsc

MAXLEN = 200
EMBED = 64
BATCH = 4096
NC, NS = 2, 16
NW = NC * NS
ROWS_W = BATCH // NW
NBUF = 4
LANES = 16
G1 = 128
G2 = MAXLEN - G1


def _body(x_hbm, tok_hbm, pos_hbm, out_hbm, idx_v, pos_v, buf, g1s, g2s, ssem):
  wid = lax.axis_index("s") * NC + lax.axis_index("c")
  rbase = wid * ROWS_W

  pltpu.sync_copy(x_hbm.at[pl.ds(rbase, ROWS_W)], idx_v)
  pltpu.sync_copy(pos_hbm, pos_v)

  def g_start(r, b):
    pltpu.async_copy(tok_hbm.at[idx_v.at[r, pl.ds(0, G1)]],
                     buf.at[b, pl.ds(0, G1)], g1s.at[b])
    pltpu.async_copy(tok_hbm.at[idx_v.at[r, pl.ds(G1, G2)]],
                     buf.at[b, pl.ds(G1, G2)], g2s.at[b])

  def g_wait(r, b):
    pltpu.make_async_copy(tok_hbm.at[idx_v.at[r, pl.ds(0, G1)]],
                          buf.at[b, pl.ds(0, G1)], g1s.at[b]).wait()
    pltpu.make_async_copy(tok_hbm.at[idx_v.at[r, pl.ds(G1, G2)]],
                          buf.at[b, pl.ds(G1, G2)], g2s.at[b]).wait()

  def s_start(r, b):
    pltpu.async_copy(buf.at[b], out_hbm.at[rbase + r], ssem.at[b])

  def s_wait(r, b):
    pltpu.make_async_copy(buf.at[b], out_hbm.at[rbase + r], ssem.at[b]).wait()

  for b in range(NBUF - 1):
    g_start(b, b)

  @pl.loop(0, ROWS_W // NBUF)
  def _(t):
    for b in range(NBUF):
      r = t * NBUF + b
      bn = (b + NBUF - 1) % NBUF
      @pl.when(r + NBUF - 1 < ROWS_W)
      def _():
        @pl.when(r >= 1)
        def _():
          s_wait(r - 1, bn)
        g_start(r + NBUF - 1, bn)

      g_wait(r, b)
      bref = buf.at[b]

      @pl.loop(0, MAXLEN // 8)
      def _(i):
        for rr in range(8):
          for j in range(EMBED // LANES):
            sl = pl.ds(j * LANES, LANES)
            plsc.addupdate(bref.at[i * 8 + rr, sl], pos_v[i * 8 + rr, sl])

      s_start(r, b)

  for b in range(NBUF):
    s_wait(ROWS_W - NBUF + b, b)


@functools.partial(
    pl.kernel,
    out_type=jax.ShapeDtypeStruct((BATCH, MAXLEN, EMBED), jnp.float32),
    mesh=plsc.VectorSubcoreMesh(
        core_axis_name="c", subcore_axis_name="s",
        num_cores=NC, num_subcores=NS),
    scratch_types=[
        pltpu.VMEM((ROWS_W, MAXLEN), jnp.int32),
        pltpu.VMEM((MAXLEN, EMBED), jnp.float32),
        pltpu.VMEM((NBUF, MAXLEN, EMBED), jnp.float32),
        pltpu.SemaphoreType.DMA((NBUF,)),
        pltpu.SemaphoreType.DMA((NBUF,)),
        pltpu.SemaphoreType.DMA((NBUF,)),
    ],
    compiler_params=pltpu.CompilerParams(use_tc_tiling_on_sc=False),
)
def _emb(x_hbm, tok_hbm, pos_hbm, out_hbm, idx_v, pos_v, buf, g1s, g2s, ssem):
  _body(x_hbm, tok_hbm, pos_hbm, out_hbm, idx_v, pos_v, buf, g1s, g2s, ssem)


@jax.jit
def kernel(x, token_table, pos_table):
  return _emb(x.astype(jnp.int32), token_table, pos_table)
